# Initial kernel scaffold; baseline (speedup 1.0000x reference)
import jax
import jax.numpy as jnp
from jax.experimental import pallas as pl


def kernel(x, edge_index, W, b):
    raise NotImplementedError("write your pallas kernel here")


# speedup vs baseline: 10.7329x; 10.7329x over previous
import functools

import jax
import jax.numpy as jnp
from jax import lax
from jax.experimental import pallas as pl
from jax.experimental.pallas import tpu as pltpu
from jax.experimental.pallas import tpu_sc as plsc

N = 10000
NPAD = 10240
RPT = 640
E = 320000
CH = 128
NCHUNK = 157
EPT = CH * NCHUNK
EPAD = EPT * 16
PAD_NODE = 10016
D = 128
F = 64
C = 64
K = 2


def _scale_rows(buf, dinv2, power):
    def body(i, _):
        dv = dinv2[i, :]
        if power == 2:
            dv = dv * dv
        for k in range(F // 16):
            sl = pl.ds(k * 16, 16)
            buf[i, sl] = buf[i, sl] * dv
        return 0
    lax.fori_loop(0, RPT, body, 0, unroll=False)


def _make_prop():
    mesh = plsc.VectorSubcoreMesh(core_axis_name="c", subcore_axis_name="s")

    @functools.partial(
        pl.kernel,
        out_type=[
            jax.ShapeDtypeStruct((NPAD, D), jnp.float32),
            jax.ShapeDtypeStruct((2 * NPAD, F), jnp.float32),
        ],
        mesh=mesh,
        compiler_params=pltpu.CompilerParams(use_tc_tiling_on_sc=False,
                                             needs_layout_passes=False),
        scratch_types=[
            pltpu.VMEM((CH,), jnp.int32),
            pltpu.VMEM((CH,), jnp.int32),
            pltpu.VMEM((CH, F), jnp.float32),
            pltpu.VMEM((RPT, F), jnp.float32),
            pltpu.VMEM((CH, 16), jnp.float32),
            pltpu.VMEM((RPT, 16), jnp.float32),
            pltpu.VMEM((RPT, 16), jnp.float32),
            pltpu.VMEM_SHARED((NPAD, F), jnp.float32),
            pltpu.VMEM_SHARED((NPAD, 16), jnp.float32),
            pltpu.SemaphoreType.DMA,
        ],
    )
    def prop(x_hbm, src2_hbm, dst_hbm, h2_hbm, s_hbm,
             idx_src, idx_dst, gbuf, xbuf, onesb, deg2, dinv2,
             a_sh, deg_sh, sem):
        c = lax.axis_index("c")
        t = lax.axis_index("s")
        row0 = t * RPT
        e0 = t * EPT

        one16 = jnp.full((16,), 1.0, dtype=jnp.float32)

        def initones(i, _):
            onesb[i, :] = one16
            return 0
        lax.fori_loop(0, CH, initones, 0, unroll=False)

        def initdeg(i, _):
            deg2[i, :] = one16
            return 0
        lax.fori_loop(0, RPT, initdeg, 0, unroll=False)
        pltpu.sync_copy(deg2, deg_sh.at[pl.ds(row0, RPT)])
        plsc.subcore_barrier()

        def degchunk(j, _):
            pltpu.sync_copy(dst_hbm.at[pl.ds(e0 + j * CH, CH)], idx_dst)
            pltpu.sync_copy(onesb, deg_sh.at[idx_dst], add=True)
            return 0
        lax.fori_loop(0, NCHUNK, degchunk, 0, unroll=False)
        plsc.subcore_barrier()

        pltpu.sync_copy(deg_sh.at[pl.ds(row0, RPT)], deg2)

        def rsqrt_row(i, _):
            v = deg2[i, :]
            yi = jnp.int32(0x5F3759DF) - (plsc.bitcast(v, jnp.int32) >> 1)
            y = plsc.bitcast(yi, jnp.float32)
            for _ in range(4):
                y = y * (1.5 - 0.5 * v * y * y)
            dinv2[i, :] = y
            return 0
        lax.fori_loop(0, RPT, rsqrt_row, 0, unroll=False)

        pltpu.sync_copy(x_hbm.at[pl.ds(row0, RPT), pl.ds(c * F, F)], xbuf)
        _scale_rows(xbuf, dinv2, 1)
        pltpu.sync_copy(xbuf, s_hbm.at[pl.ds(c * NPAD + row0, RPT)])
        pltpu.sync_copy(xbuf, a_sh.at[pl.ds(row0, RPT)])
        plsc.subcore_barrier()

        def edgechunk(j, _):
            pltpu.sync_copy(src2_hbm.at[c, pl.ds(e0 + j * CH, CH)], idx_src)
            pltpu.sync_copy(dst_hbm.at[pl.ds(e0 + j * CH, CH)], idx_dst)
            pltpu.async_copy(s_hbm.at[idx_src], gbuf, sem).wait()
            pltpu.sync_copy(gbuf, a_sh.at[idx_dst], add=True)
            return 0

        for r in range(K):
            lax.fori_loop(0, NCHUNK, edgechunk, 0, unroll=False)
            plsc.subcore_barrier()
            pltpu.sync_copy(a_sh.at[pl.ds(row0, RPT)], xbuf)
            if r < K - 1:
                _scale_rows(xbuf, dinv2, 2)
                pltpu.sync_copy(xbuf, s_hbm.at[pl.ds(c * NPAD + row0, RPT)])
                pltpu.sync_copy(xbuf, a_sh.at[pl.ds(row0, RPT)])
                plsc.subcore_barrier()
            else:
                _scale_rows(xbuf, dinv2, 1)
                pltpu.sync_copy(xbuf, h2_hbm.at[pl.ds(row0, RPT), pl.ds(c * F, F)])

    return prop


_prop = _make_prop()


def _head_body(h_ref, w_ref, b_ref, o_ref):
    z = lax.dot_general(h_ref[...], w_ref[...], (((1,), (1,)), ((), ())),
                        preferred_element_type=jnp.float32)
    z = z + b_ref[...]
    m = jnp.max(z, axis=-1, keepdims=True)
    lse = jnp.log(jnp.sum(jnp.exp(z - m), axis=-1, keepdims=True))
    o_ref[...] = z - m - lse


_head = pl.pallas_call(
    _head_body,
    grid=(10,),
    in_specs=[
        pl.BlockSpec((N // 10, D), lambda i: (i, 0)),
        pl.BlockSpec((C, D), lambda i: (0, 0)),
        pl.BlockSpec((1, C), lambda i: (0, 0)),
    ],
    out_specs=pl.BlockSpec((N // 10, C), lambda i: (i, 0)),
    out_shape=jax.ShapeDtypeStruct((N, C), jnp.float32),
)


def kernel(x, edge_index, W, b):
    src = edge_index[0].astype(jnp.int32)
    dst = edge_index[1].astype(jnp.int32)
    pad = EPAD - E
    padv = jnp.full((pad,), PAD_NODE, jnp.int32)
    src = jnp.concatenate([src, padv])
    dst = jnp.concatenate([dst, padv])
    src2 = jnp.stack([src, src + NPAD])
    xp = jnp.zeros((NPAD, D), jnp.float32).at[:N].set(x)
    h2, _ = _prop(xp, src2, dst)
    return _head(h2[:N], W, b.reshape(1, C))


# speedup vs baseline: 15.1589x; 1.4124x over previous
import functools

import jax
import jax.numpy as jnp
from jax import lax
from jax.experimental import pallas as pl
from jax.experimental.pallas import tpu as pltpu
from jax.experimental.pallas import tpu_sc as plsc

N = 10000
NPAD = 10240
RPT = 640
HRPT = 320
E = 320000
CH = 128
NCHUNK = 160
EPT = CH * NCHUNK
EPAD = EPT * 16
PAD_NODE = 10016
D = 128
F = 64
C = 64
K = 2
NB = 4
NI = 8


def _make_prop():
    mesh = plsc.VectorSubcoreMesh(core_axis_name="c", subcore_axis_name="s")

    @functools.partial(
        pl.kernel,
        out_type=[
            jax.ShapeDtypeStruct((NPAD, D), jnp.float32),
            jax.ShapeDtypeStruct((2 * NPAD, F), jnp.float32),
        ],
        mesh=mesh,
        compiler_params=pltpu.CompilerParams(use_tc_tiling_on_sc=False,
                                             needs_layout_passes=False),
        scratch_types=[
            pltpu.VMEM((NI, CH), jnp.int32),
            pltpu.VMEM((NI, CH), jnp.int32),
            pltpu.VMEM((NB, CH, F), jnp.float32),
            pltpu.VMEM((HRPT, F), jnp.float32),
            pltpu.VMEM((CH, 16), jnp.float32),
            pltpu.VMEM((RPT, 16), jnp.float32),
            pltpu.VMEM_SHARED((NPAD, F), jnp.float32),
            pltpu.VMEM_SHARED((NPAD, 16), jnp.float32),
            pltpu.SemaphoreType.DMA,
            pltpu.SemaphoreType.DMA,
            pltpu.SemaphoreType.DMA,
            pltpu.SemaphoreType.DMA,
            pltpu.SemaphoreType.DMA,
            pltpu.SemaphoreType.DMA,
            pltpu.SemaphoreType.DMA,
            pltpu.SemaphoreType.DMA,
            pltpu.SemaphoreType.DMA,
            pltpu.SemaphoreType.DMA,
            pltpu.SemaphoreType.DMA,
            pltpu.SemaphoreType.DMA,
        ],
    )
    def prop(x_hbm, src3_hbm, dst3_hbm, h2_hbm, s_hbm,
             isrc, idst, gbufs, xbuf, onesb, deg2,
             a_sh, deg_sh,
             g0, g1, g2, g3, i0, i1, i2, i3, i4, i5, i6, i7):
        c = lax.axis_index("c")
        t = lax.axis_index("s")
        row0 = t * RPT
        sems_g = [g0, g1, g2, g3]
        sems_i = [i0, i1, i2, i3, i4, i5, i6, i7]

        def didx_issue(cur, u):
            pltpu.async_copy(dst3_hbm.at[t, cur], idst.at[u], sems_i[u])

        def didx_wait(u):
            pltpu.make_async_copy(dst3_hbm.at[t, 0], idst.at[u],
                                  sems_i[u]).wait()

        def sidx_issue(cur, u):
            pltpu.async_copy(src3_hbm.at[c, t, cur], isrc.at[u], sems_i[u])

        def sidx_wait(u):
            pltpu.make_async_copy(src3_hbm.at[c, t, 0], isrc.at[u],
                                  sems_i[u]).wait()

        def gather_issue(u, b):
            pltpu.async_copy(s_hbm.at[isrc.at[u]], gbufs.at[b], sems_g[b])

        def gather_wait(u, b):
            pltpu.make_async_copy(s_hbm.at[isrc.at[u]], gbufs.at[b],
                                  sems_g[b]).wait()

        def scatter(u, b):
            pltpu.sync_copy(gbufs.at[b], a_sh.at[idst.at[u]], add=True)

        one16 = jnp.full((16,), 1.0, dtype=jnp.float32)

        def initones(i, _):
            onesb[i, :] = one16
            return 0
        lax.fori_loop(0, CH, initones, 0, unroll=False)

        def initdeg(i, _):
            deg2[i, :] = one16
            return 0
        lax.fori_loop(0, RPT, initdeg, 0, unroll=False)
        pltpu.sync_copy(deg2.at[pl.ds(0, HRPT)], deg_sh.at[pl.ds(row0, HRPT)])
        pltpu.sync_copy(deg2.at[pl.ds(HRPT, HRPT)],
                        deg_sh.at[pl.ds(row0 + HRPT, HRPT)])
        plsc.subcore_barrier()

        for u in range(NI):
            didx_issue(u, u)

        def deg_steady(i, _):
            j = i * NI
            for u in range(NI):
                didx_wait(u)
                pltpu.sync_copy(onesb, deg_sh.at[idst.at[u]], add=True)
                didx_issue(j + u + NI, u)
            return 0
        lax.fori_loop(0, NCHUNK // NI - 1, deg_steady, 0, unroll=False)
        for u in range(NI):
            didx_wait(u)
            pltpu.sync_copy(onesb, deg_sh.at[idst.at[u]], add=True)
        plsc.subcore_barrier()

        pltpu.sync_copy(deg_sh.at[pl.ds(row0, HRPT)], deg2.at[pl.ds(0, HRPT)])
        pltpu.sync_copy(deg_sh.at[pl.ds(row0 + HRPT, HRPT)],
                        deg2.at[pl.ds(HRPT, HRPT)])

        def rsqrt_row(i, _):
            v = deg2[i, :]
            yi = jnp.int32(0x5F3759DF) - (plsc.bitcast(v, jnp.int32) >> 1)
            y = plsc.bitcast(yi, jnp.float32)
            for _ in range(4):
                y = y * (1.5 - 0.5 * v * y * y)
            deg2[i, :] = y
            return 0
        lax.fori_loop(0, RPT, rsqrt_row, 0, unroll=False)

        def scale_half(p, power):
            def body(i, _):
                dv = deg2[p * HRPT + i, :]
                if power == 2:
                    dv = dv * dv
                for k in range(F // 16):
                    sl = pl.ds(k * 16, 16)
                    xbuf[i, sl] = xbuf[i, sl] * dv
                return 0
            lax.fori_loop(0, HRPT, body, 0, unroll=False)

        for p in range(2):
            rp = row0 + p * HRPT
            pltpu.sync_copy(x_hbm.at[pl.ds(rp, HRPT), pl.ds(c * F, F)], xbuf)
            scale_half(p, 1)
            pltpu.sync_copy(xbuf, s_hbm.at[pl.ds(c * NPAD + rp, HRPT)])
            pltpu.sync_copy(xbuf, a_sh.at[pl.ds(rp, HRPT)])
        plsc.subcore_barrier()

        for r in range(K):
            for u in range(NI):
                sidx_issue(u, u)
                didx_issue(u, u)
            for u in range(NB):
                sidx_wait(u)
                didx_wait(u)
                gather_issue(u, u)

            def steady(i, _):
                j = i * NI
                for u in range(NI):
                    b = u % NB
                    u4 = (u + NB) % NI
                    gather_wait(u, b)
                    scatter(u, b)
                    sidx_issue(j + u + NI, u)
                    didx_issue(j + u + NI, u)
                    sidx_wait(u4)
                    didx_wait(u4)
                    gather_issue(u4, b)
                return 0
            lax.fori_loop(0, NCHUNK // NI - 1, steady, 0, unroll=False)

            for u in range(NB):
                b = u % NB
                u4 = (u + NB) % NI
                gather_wait(u, b)
                scatter(u, b)
                sidx_wait(u4)
                didx_wait(u4)
                gather_issue(u4, b)
            for u in range(NB, NI):
                b = u % NB
                gather_wait(u, b)
                scatter(u, b)

            plsc.subcore_barrier()
            for p in range(2):
                rp = row0 + p * HRPT
                pltpu.sync_copy(a_sh.at[pl.ds(rp, HRPT)], xbuf)
                if r < K - 1:
                    scale_half(p, 2)
                    pltpu.sync_copy(xbuf, s_hbm.at[pl.ds(c * NPAD + rp, HRPT)])
                    pltpu.sync_copy(xbuf, a_sh.at[pl.ds(rp, HRPT)])
                else:
                    scale_half(p, 1)
                    pltpu.sync_copy(xbuf, h2_hbm.at[pl.ds(rp, HRPT),
                                                    pl.ds(c * F, F)])
            if r < K - 1:
                plsc.subcore_barrier()

    return prop


_prop = _make_prop()


def _head_body(h_ref, w_ref, b_ref, o_ref):
    z = lax.dot_general(h_ref[...], w_ref[...], (((1,), (1,)), ((), ())),
                        preferred_element_type=jnp.float32)
    z = z + b_ref[...]
    m = jnp.max(z, axis=-1, keepdims=True)
    lse = jnp.log(jnp.sum(jnp.exp(z - m), axis=-1, keepdims=True))
    o_ref[...] = z - m - lse


_head = pl.pallas_call(
    _head_body,
    grid=(10,),
    in_specs=[
        pl.BlockSpec((N // 10, D), lambda i: (i, 0)),
        pl.BlockSpec((C, D), lambda i: (0, 0)),
        pl.BlockSpec((1, C), lambda i: (0, 0)),
    ],
    out_specs=pl.BlockSpec((N // 10, C), lambda i: (i, 0)),
    out_shape=jax.ShapeDtypeStruct((N, C), jnp.float32),
)


def kernel(x, edge_index, W, b):
    src = edge_index[0].astype(jnp.int32)
    dst = edge_index[1].astype(jnp.int32)
    pad = EPAD - E
    padv = jnp.full((pad,), PAD_NODE, jnp.int32)
    src = jnp.concatenate([src, padv])
    dst = jnp.concatenate([dst, padv])
    src3 = jnp.stack([src, src + NPAD]).reshape(2, 16, NCHUNK, CH)
    dst3 = dst.reshape(16, NCHUNK, CH)
    xp = jnp.zeros((NPAD, D), jnp.float32).at[:N].set(x)
    h2, _ = _prop(xp, src3, dst3)
    return _head(h2[:N], W, b.reshape(1, C))


# speedup vs baseline: 22.5835x; 1.4898x over previous
import functools

import jax
import jax.numpy as jnp
from jax import lax
from jax.experimental import pallas as pl
from jax.experimental.pallas import tpu as pltpu
from jax.experimental.pallas import tpu_sc as plsc

N = 10000
NPAD = 10240
RPT = 640
QRPT = 160
E = 320000
CH = 128
NCHUNK = 160
EPT = CH * NCHUNK
EPAD = EPT * 16
PAD_NODE = 10016
D = 128
F = 64
C = 64
K = 2
NB = 2
NI = 8


def _make_prop():
    mesh = plsc.VectorSubcoreMesh(core_axis_name="c", subcore_axis_name="s")

    @functools.partial(
        pl.kernel,
        out_type=[
            jax.ShapeDtypeStruct((NPAD, D), jnp.float32),
        ],
        mesh=mesh,
        compiler_params=pltpu.CompilerParams(use_tc_tiling_on_sc=False,
                                             needs_layout_passes=False),
        scratch_types=[
            pltpu.VMEM((NI, CH), jnp.int32),
            pltpu.VMEM((NI, CH), jnp.int32),
            pltpu.VMEM((NB, CH, F), jnp.float32),
            pltpu.VMEM((QRPT, F), jnp.float32),
            pltpu.VMEM((CH, F), jnp.float32),
            pltpu.VMEM((RPT, 16), jnp.float32),
            pltpu.VMEM_SHARED((NPAD, F), jnp.float32),
            pltpu.VMEM_SHARED((NPAD, F), jnp.float32),
            pltpu.SemaphoreType.DMA,
            pltpu.SemaphoreType.DMA,
            pltpu.SemaphoreType.DMA,
            pltpu.SemaphoreType.DMA,
            pltpu.SemaphoreType.DMA,
            pltpu.SemaphoreType.DMA,
            pltpu.SemaphoreType.DMA,
            pltpu.SemaphoreType.DMA,
            pltpu.SemaphoreType.DMA,
            pltpu.SemaphoreType.DMA,
        ],
    )
    def prop(x_hbm, src2_hbm, dst3_hbm, h2_hbm,
             isrc, idst, gbufs, xbuf, onesb, dinv2,
             a_sh, s_sh,
             g0, g1, i0, i1, i2, i3, i4, i5, i6, i7):
        c = lax.axis_index("c")
        t = lax.axis_index("s")
        row0 = t * RPT
        sems_g = [g0, g1]
        sems_i = [i0, i1, i2, i3, i4, i5, i6, i7]

        def didx_issue(cur, u):
            pltpu.async_copy(dst3_hbm.at[t, cur], idst.at[u], sems_i[u])

        def didx_wait(u):
            pltpu.make_async_copy(dst3_hbm.at[t, 0], idst.at[u],
                                  sems_i[u]).wait()

        def sidx_issue(cur, u):
            pltpu.async_copy(src2_hbm.at[t, cur], isrc.at[u], sems_i[u])

        def sidx_wait(u):
            pltpu.make_async_copy(src2_hbm.at[t, 0], isrc.at[u],
                                  sems_i[u]).wait()

        def gather_issue(u, b):
            pltpu.async_copy(s_sh.at[isrc.at[u]], gbufs.at[b], sems_g[b])

        def gather_wait(u, b):
            pltpu.make_async_copy(s_sh.at[isrc.at[u]], gbufs.at[b],
                                  sems_g[b]).wait()

        def scatter(u, b):
            pltpu.sync_copy(gbufs.at[b], a_sh.at[idst.at[u]], add=True)

        one16 = jnp.full((16,), 1.0, dtype=jnp.float32)

        def initones(i, _):
            for k in range(F // 16):
                onesb[i, pl.ds(k * 16, 16)] = one16
            return 0
        lax.fori_loop(0, CH, initones, 0, unroll=False)
        for q in range(RPT // CH):
            pltpu.sync_copy(onesb, a_sh.at[pl.ds(row0 + q * CH, CH)])
        plsc.subcore_barrier()

        for u in range(NI):
            didx_issue(u, u)

        def deg_steady(i, _):
            j = i * NI
            for u in range(NI):
                didx_wait(u)
                pltpu.sync_copy(onesb, a_sh.at[idst.at[u]], add=True)
                didx_issue(j + u + NI, u)
            return 0
        lax.fori_loop(0, NCHUNK // NI - 1, deg_steady, 0, unroll=False)
        for u in range(NI):
            didx_wait(u)
            pltpu.sync_copy(onesb, a_sh.at[idst.at[u]], add=True)
        plsc.subcore_barrier()

        for p in range(RPT // QRPT):
            pltpu.sync_copy(a_sh.at[pl.ds(row0 + p * QRPT, QRPT)], xbuf)

            def rsqrt_row(i, _):
                v = xbuf[i, pl.ds(0, 16)]
                yi = jnp.int32(0x5F3759DF) - (plsc.bitcast(v, jnp.int32) >> 1)
                y = plsc.bitcast(yi, jnp.float32)
                for _ in range(4):
                    y = y * (1.5 - 0.5 * v * y * y)
                dinv2[p * QRPT + i, :] = y
                return 0
            lax.fori_loop(0, QRPT, rsqrt_row, 0, unroll=False)

        def scale_quarter(p, power):
            def body(i, _):
                dv = dinv2[p * QRPT + i, :]
                if power == 2:
                    dv = dv * dv
                for k in range(F // 16):
                    sl = pl.ds(k * 16, 16)
                    xbuf[i, sl] = xbuf[i, sl] * dv
                return 0
            lax.fori_loop(0, QRPT, body, 0, unroll=False)

        for p in range(RPT // QRPT):
            rp = row0 + p * QRPT
            pltpu.sync_copy(x_hbm.at[pl.ds(rp, QRPT), pl.ds(c * F, F)], xbuf)
            scale_quarter(p, 1)
            pltpu.sync_copy(xbuf, s_sh.at[pl.ds(rp, QRPT)])
            pltpu.sync_copy(xbuf, a_sh.at[pl.ds(rp, QRPT)])
        plsc.subcore_barrier()

        for r in range(K):
            for u in range(NI):
                sidx_issue(u, u)
                didx_issue(u, u)
            for u in range(NB):
                sidx_wait(u)
                didx_wait(u)
                gather_issue(u, u)

            def steady(i, _):
                j = i * NI
                for u in range(NI):
                    b = u % NB
                    un = (u + NB) % NI
                    gather_wait(u, b)
                    scatter(u, b)
                    sidx_issue(j + u + NI, u)
                    didx_issue(j + u + NI, u)
                    sidx_wait(un)
                    didx_wait(un)
                    gather_issue(un, un % NB)
                return 0
            lax.fori_loop(0, NCHUNK // NI - 1, steady, 0, unroll=False)

            for u in range(NI):
                b = u % NB
                gather_wait(u, b)
                scatter(u, b)
                un = u + NB
                if un < NI:
                    sidx_wait(un)
                    didx_wait(un)
                    gather_issue(un, un % NB)

            plsc.subcore_barrier()
            for p in range(RPT // QRPT):
                rp = row0 + p * QRPT
                pltpu.sync_copy(a_sh.at[pl.ds(rp, QRPT)], xbuf)
                if r < K - 1:
                    scale_quarter(p, 2)
                    pltpu.sync_copy(xbuf, s_sh.at[pl.ds(rp, QRPT)])
                    pltpu.sync_copy(xbuf, a_sh.at[pl.ds(rp, QRPT)])
                else:
                    scale_quarter(p, 1)
                    pltpu.sync_copy(xbuf, h2_hbm.at[pl.ds(rp, QRPT),
                                                    pl.ds(c * F, F)])
            if r < K - 1:
                plsc.subcore_barrier()

    return prop


_prop = _make_prop()


def _head_body(h_ref, w_ref, b_ref, o_ref):
    z = lax.dot_general(h_ref[...], w_ref[...], (((1,), (1,)), ((), ())),
                        preferred_element_type=jnp.float32)
    z = z + b_ref[...]
    m = jnp.max(z, axis=-1, keepdims=True)
    lse = jnp.log(jnp.sum(jnp.exp(z - m), axis=-1, keepdims=True))
    o_ref[...] = z - m - lse


_head = pl.pallas_call(
    _head_body,
    grid=(10,),
    in_specs=[
        pl.BlockSpec((N // 10, D), lambda i: (i, 0)),
        pl.BlockSpec((C, D), lambda i: (0, 0)),
        pl.BlockSpec((1, C), lambda i: (0, 0)),
    ],
    out_specs=pl.BlockSpec((N // 10, C), lambda i: (i, 0)),
    out_shape=jax.ShapeDtypeStruct((N, C), jnp.float32),
)


def kernel(x, edge_index, W, b):
    src = edge_index[0].astype(jnp.int32)
    dst = edge_index[1].astype(jnp.int32)
    pad = EPAD - E
    padv = jnp.full((pad,), PAD_NODE, jnp.int32)
    src = jnp.concatenate([src, padv])
    dst = jnp.concatenate([dst, padv])
    src2 = src.reshape(16, NCHUNK, CH)
    dst3 = dst.reshape(16, NCHUNK, CH)
    xp = jnp.zeros((NPAD, D), jnp.float32).at[:N].set(x)
    (h2,) = _prop(xp, src2, dst3)
    return _head(h2[:N], W, b.reshape(1, C))


# speedup vs baseline: 24.9481x; 1.1047x over previous
import functools

import jax
import jax.numpy as jnp
from jax import lax
from jax.experimental import pallas as pl
from jax.experimental.pallas import tpu as pltpu
from jax.experimental.pallas import tpu_sc as plsc

N = 10000
NPAD = 10240
RPT = 640
QRPT = 80
E = 320000
CH = 128
NCHUNK = 160
EPT = CH * NCHUNK
EPAD = EPT * 16
PAD_NODE = 10016
D = 128
F = 64
C = 64
K = 2
NB = 2
NI = 8


def _make_prop():
    mesh = plsc.VectorSubcoreMesh(core_axis_name="c", subcore_axis_name="s")

    @functools.partial(
        pl.kernel,
        out_type=[
            jax.ShapeDtypeStruct((NPAD, D), jnp.float32),
        ],
        mesh=mesh,
        compiler_params=pltpu.CompilerParams(use_tc_tiling_on_sc=False,
                                             needs_layout_passes=False),
        scratch_types=[
            pltpu.VMEM((NI, CH), jnp.int32),
            pltpu.VMEM((NI, CH), jnp.int32),
            pltpu.VMEM((NB, CH, F), jnp.float32),
            pltpu.VMEM((QRPT, F), jnp.float32),
            pltpu.VMEM((CH, 16), jnp.float32),
            pltpu.VMEM((RPT, 16), jnp.float32),
            pltpu.VMEM_SHARED((NPAD, F), jnp.float32),
            pltpu.VMEM_SHARED((NPAD, F), jnp.float32),
            pltpu.VMEM_SHARED((NPAD, 16), jnp.float32),
            pltpu.SemaphoreType.DMA,
            pltpu.SemaphoreType.DMA,
            pltpu.SemaphoreType.DMA,
            pltpu.SemaphoreType.DMA,
            pltpu.SemaphoreType.DMA,
            pltpu.SemaphoreType.DMA,
            pltpu.SemaphoreType.DMA,
            pltpu.SemaphoreType.DMA,
            pltpu.SemaphoreType.DMA,
            pltpu.SemaphoreType.DMA,
        ],
    )
    def prop(x_hbm, src2_hbm, dst3_hbm, h2_hbm,
             isrc, idst, gbufs, xbuf, onesb, dinv2,
             a_sh, s_sh, deg_sh,
             g0, g1, i0, i1, i2, i3, i4, i5, i6, i7):
        c = lax.axis_index("c")
        t = lax.axis_index("s")
        row0 = t * RPT
        sems_g = [g0, g1]
        sems_i = [i0, i1, i2, i3, i4, i5, i6, i7]

        def didx_issue(cur, u):
            pltpu.async_copy(dst3_hbm.at[t, cur], idst.at[u], sems_i[u])

        def didx_wait(u):
            pltpu.make_async_copy(dst3_hbm.at[t, 0], idst.at[u],
                                  sems_i[u]).wait()

        def sidx_issue(cur, u):
            pltpu.async_copy(src2_hbm.at[t, cur], isrc.at[u], sems_i[u])

        def sidx_wait(u):
            pltpu.make_async_copy(src2_hbm.at[t, 0], isrc.at[u],
                                  sems_i[u]).wait()

        def gather_issue(u, b):
            pltpu.async_copy(s_sh.at[isrc.at[u]], gbufs.at[b], sems_g[b])

        def gather_wait(u, b):
            pltpu.make_async_copy(s_sh.at[isrc.at[u]], gbufs.at[b],
                                  sems_g[b]).wait()

        def scatter(u, b):
            pltpu.sync_copy(gbufs.at[b], a_sh.at[idst.at[u]], add=True)

        one16 = jnp.full((16,), 1.0, dtype=jnp.float32)

        def initones(i, _):
            onesb[i, :] = one16
            return 0
        lax.fori_loop(0, CH, initones, 0, unroll=False)
        for q in range(RPT // CH):
            pltpu.sync_copy(onesb, deg_sh.at[pl.ds(row0 + q * CH, CH)])
        plsc.subcore_barrier()

        for u in range(NI):
            didx_issue(u, u)

        def deg_steady(i, _):
            j = i * NI
            for u in range(NI):
                didx_wait(u)
                pltpu.sync_copy(onesb, deg_sh.at[idst.at[u]], add=True)
                didx_issue(j + u + NI, u)
            return 0
        lax.fori_loop(0, NCHUNK // NI - 1, deg_steady, 0, unroll=False)
        for u in range(NI):
            didx_wait(u)
            pltpu.sync_copy(onesb, deg_sh.at[idst.at[u]], add=True)
        plsc.subcore_barrier()

        pltpu.sync_copy(deg_sh.at[pl.ds(row0, RPT)], dinv2)

        def rsqrt_row(i, _):
            v = dinv2[i, :]
            yi = jnp.int32(0x5F3759DF) - (plsc.bitcast(v, jnp.int32) >> 1)
            y = plsc.bitcast(yi, jnp.float32)
            for _ in range(4):
                y = y * (1.5 - 0.5 * v * y * y)
            dinv2[i, :] = y
            return 0
        lax.fori_loop(0, RPT, rsqrt_row, 0, unroll=False)

        def scale_quarter(p, power):
            def body(i, _):
                dv = dinv2[p * QRPT + i, :]
                if power == 2:
                    dv = dv * dv
                for k in range(F // 16):
                    sl = pl.ds(k * 16, 16)
                    xbuf[i, sl] = xbuf[i, sl] * dv
                return 0
            lax.fori_loop(0, QRPT, body, 0, unroll=False)

        for p in range(RPT // QRPT):
            rp = row0 + p * QRPT
            pltpu.sync_copy(x_hbm.at[pl.ds(rp, QRPT), pl.ds(c * F, F)], xbuf)
            scale_quarter(p, 1)
            pltpu.sync_copy(xbuf, s_sh.at[pl.ds(rp, QRPT)])
            pltpu.sync_copy(xbuf, a_sh.at[pl.ds(rp, QRPT)])
        plsc.subcore_barrier()

        for r in range(K):
            for u in range(NI):
                sidx_issue(u, u)
                didx_issue(u, u)
            for u in range(NB):
                sidx_wait(u)
                didx_wait(u)
                gather_issue(u, u)

            def steady(i, _):
                j = i * NI
                for u in range(NI):
                    b = u % NB
                    un = (u + NB) % NI
                    gather_wait(u, b)
                    scatter(u, b)
                    sidx_issue(j + u + NI, u)
                    didx_issue(j + u + NI, u)
                    sidx_wait(un)
                    didx_wait(un)
                    gather_issue(un, un % NB)
                return 0
            lax.fori_loop(0, NCHUNK // NI - 1, steady, 0, unroll=False)

            for u in range(NI):
                b = u % NB
                gather_wait(u, b)
                scatter(u, b)
                un = u + NB
                if un < NI:
                    sidx_wait(un)
                    didx_wait(un)
                    gather_issue(un, un % NB)

            plsc.subcore_barrier()
            for p in range(RPT // QRPT):
                rp = row0 + p * QRPT
                pltpu.sync_copy(a_sh.at[pl.ds(rp, QRPT)], xbuf)
                if r < K - 1:
                    scale_quarter(p, 2)
                    pltpu.sync_copy(xbuf, s_sh.at[pl.ds(rp, QRPT)])
                    pltpu.sync_copy(xbuf, a_sh.at[pl.ds(rp, QRPT)])
                else:
                    scale_quarter(p, 1)
                    pltpu.sync_copy(xbuf, h2_hbm.at[pl.ds(rp, QRPT),
                                                    pl.ds(c * F, F)])
            if r < K - 1:
                plsc.subcore_barrier()

    return prop


_prop = _make_prop()


def _head_body(h_ref, w_ref, b_ref, o_ref):
    z = lax.dot_general(h_ref[...], w_ref[...], (((1,), (1,)), ((), ())),
                        preferred_element_type=jnp.float32)
    z = z + b_ref[...]
    m = jnp.max(z, axis=-1, keepdims=True)
    lse = jnp.log(jnp.sum(jnp.exp(z - m), axis=-1, keepdims=True))
    o_ref[...] = z - m - lse


_head = pl.pallas_call(
    _head_body,
    grid=(10,),
    in_specs=[
        pl.BlockSpec((N // 10, D), lambda i: (i, 0)),
        pl.BlockSpec((C, D), lambda i: (0, 0)),
        pl.BlockSpec((1, C), lambda i: (0, 0)),
    ],
    out_specs=pl.BlockSpec((N // 10, C), lambda i: (i, 0)),
    out_shape=jax.ShapeDtypeStruct((N, C), jnp.float32),
)


def kernel(x, edge_index, W, b):
    src = edge_index[0].astype(jnp.int32)
    dst = edge_index[1].astype(jnp.int32)
    pad = EPAD - E
    padv = jnp.full((pad,), PAD_NODE, jnp.int32)
    src = jnp.concatenate([src, padv])
    dst = jnp.concatenate([dst, padv])
    src2 = src.reshape(16, NCHUNK, CH)
    dst3 = dst.reshape(16, NCHUNK, CH)
    xp = jnp.zeros((NPAD, D), jnp.float32).at[:N].set(x)
    (h2,) = _prop(xp, src2, dst3)
    return _head(h2, W, b.reshape(1, C))


# speedup vs baseline: 24.9628x; 1.0006x over previous
import functools

import jax
import jax.numpy as jnp
from jax import lax
from jax.experimental import pallas as pl
from jax.experimental.pallas import tpu as pltpu
from jax.experimental.pallas import tpu_sc as plsc

N = 10000
NPAD = 10240
RPT = 640
QRPT = 80
E = 320000
CH = 128
NCHUNK = 160
EPT = CH * NCHUNK
EPAD = EPT * 16
PAD_NODE = 10016
D = 128
F = 64
C = 64
K = 2
NB = 2
NI = 8


def _make_prop():
    mesh = plsc.VectorSubcoreMesh(core_axis_name="c", subcore_axis_name="s")

    @functools.partial(
        pl.kernel,
        out_type=[
            jax.ShapeDtypeStruct((NPAD, D), jnp.float32),
        ],
        mesh=mesh,
        compiler_params=pltpu.CompilerParams(use_tc_tiling_on_sc=False,
                                             needs_layout_passes=False),
        scratch_types=[
            pltpu.VMEM((NI, CH), jnp.int32),
            pltpu.VMEM((NI, CH), jnp.int32),
            pltpu.VMEM((NB, CH, F), jnp.float32),
            pltpu.VMEM((QRPT, F), jnp.float32),
            pltpu.VMEM((CH, 16), jnp.float32),
            pltpu.VMEM((RPT, 16), jnp.float32),
            pltpu.VMEM_SHARED((NPAD, F), jnp.float32),
            pltpu.VMEM_SHARED((NPAD, F), jnp.float32),
            pltpu.VMEM_SHARED((NPAD, 16), jnp.float32),
            pltpu.SemaphoreType.DMA,
            pltpu.SemaphoreType.DMA,
            pltpu.SemaphoreType.DMA,
            pltpu.SemaphoreType.DMA,
            pltpu.SemaphoreType.DMA,
            pltpu.SemaphoreType.DMA,
            pltpu.SemaphoreType.DMA,
            pltpu.SemaphoreType.DMA,
            pltpu.SemaphoreType.DMA,
            pltpu.SemaphoreType.DMA,
            pltpu.SemaphoreType.DMA,
            pltpu.SemaphoreType.DMA,
        ],
    )
    def prop(x_hbm, src2_hbm, dst3_hbm, h2_hbm,
             isrc, idst, gbufs, xbuf, onesb, dinv2,
             a_sh, s_sh, deg_sh,
             g0, g1, s0, s1, i0, i1, i2, i3, i4, i5, i6, i7):
        c = lax.axis_index("c")
        t = lax.axis_index("s")
        row0 = t * RPT
        sems_g = [g0, g1]
        sems_s = [s0, s1]
        sems_i = [i0, i1, i2, i3, i4, i5, i6, i7]

        def didx_issue(cur, u):
            pltpu.async_copy(dst3_hbm.at[t, cur], idst.at[u], sems_i[u])

        def didx_wait(u):
            pltpu.make_async_copy(dst3_hbm.at[t, 0], idst.at[u],
                                  sems_i[u]).wait()

        def sidx_issue(cur, u):
            pltpu.async_copy(src2_hbm.at[t, cur], isrc.at[u], sems_i[u])

        def sidx_wait(u):
            pltpu.make_async_copy(src2_hbm.at[t, 0], isrc.at[u],
                                  sems_i[u]).wait()

        def gather_issue(u, b):
            pltpu.async_copy(s_sh.at[isrc.at[u]], gbufs.at[b], sems_g[b])

        def gather_wait(u, b):
            pltpu.make_async_copy(s_sh.at[isrc.at[u]], gbufs.at[b],
                                  sems_g[b]).wait()

        def scatter(u, b):
            pltpu.sync_copy(gbufs.at[b], a_sh.at[idst.at[u]], add=True)

        one16 = jnp.full((16,), 1.0, dtype=jnp.float32)

        def initones(i, _):
            onesb[i, :] = one16
            return 0
        lax.fori_loop(0, CH, initones, 0, unroll=False)
        for q in range(RPT // CH):
            pltpu.sync_copy(onesb, deg_sh.at[pl.ds(row0 + q * CH, CH)])
        plsc.subcore_barrier()

        for u in range(NI):
            didx_issue(u, u)

        def deg_steady(i, _):
            j = i * NI
            for u in range(NI):
                didx_wait(u)
                pltpu.sync_copy(onesb, deg_sh.at[idst.at[u]], add=True)
                didx_issue(j + u + NI, u)
            return 0
        lax.fori_loop(0, NCHUNK // NI - 1, deg_steady, 0, unroll=False)
        for u in range(NI):
            didx_wait(u)
            pltpu.sync_copy(onesb, deg_sh.at[idst.at[u]], add=True)
        plsc.subcore_barrier()

        pltpu.sync_copy(deg_sh.at[pl.ds(row0, RPT)], dinv2)

        def rsqrt_row(i, _):
            v = dinv2[i, :]
            yi = jnp.int32(0x5F3759DF) - (plsc.bitcast(v, jnp.int32) >> 1)
            y = plsc.bitcast(yi, jnp.float32)
            for _ in range(4):
                y = y * (1.5 - 0.5 * v * y * y)
            dinv2[i, :] = y
            return 0
        lax.fori_loop(0, RPT, rsqrt_row, 0, unroll=False)

        def scale_quarter(p, power):
            def body(i, _):
                dv = dinv2[p * QRPT + i, :]
                if power == 2:
                    dv = dv * dv
                for k in range(F // 16):
                    sl = pl.ds(k * 16, 16)
                    xbuf[i, sl] = xbuf[i, sl] * dv
                return 0
            lax.fori_loop(0, QRPT, body, 0, unroll=False)

        for p in range(RPT // QRPT):
            rp = row0 + p * QRPT
            pltpu.sync_copy(x_hbm.at[pl.ds(rp, QRPT), pl.ds(c * F, F)], xbuf)
            scale_quarter(p, 1)
            pltpu.sync_copy(xbuf, s_sh.at[pl.ds(rp, QRPT)])
            pltpu.sync_copy(xbuf, a_sh.at[pl.ds(rp, QRPT)])
        plsc.subcore_barrier()

        for r in range(K):
            for u in range(NI):
                sidx_issue(u, u)
                didx_issue(u, u)
            for u in range(NB):
                sidx_wait(u)
                didx_wait(u)
                gather_issue(u, u)

            def steady(i, _):
                j = i * NI
                for u in range(NI):
                    b = u % NB
                    un = (u + NB) % NI
                    gather_wait(u, b)
                    scatter(u, b)
                    sidx_issue(j + u + NI, u)
                    didx_issue(j + u + NI, u)
                    sidx_wait(un)
                    didx_wait(un)
                    gather_issue(un, un % NB)
                return 0
            lax.fori_loop(0, NCHUNK // NI - 1, steady, 0, unroll=False)

            for u in range(NI):
                b = u % NB
                gather_wait(u, b)
                scatter(u, b)
                un = u + NB
                if un < NI:
                    sidx_wait(un)
                    didx_wait(un)
                    gather_issue(un, un % NB)

            plsc.subcore_barrier()
            for p in range(RPT // QRPT):
                rp = row0 + p * QRPT
                pltpu.sync_copy(a_sh.at[pl.ds(rp, QRPT)], xbuf)
                if r < K - 1:
                    scale_quarter(p, 2)
                    pltpu.sync_copy(xbuf, s_sh.at[pl.ds(rp, QRPT)])
                    pltpu.sync_copy(xbuf, a_sh.at[pl.ds(rp, QRPT)])
                else:
                    scale_quarter(p, 1)
                    pltpu.sync_copy(xbuf, h2_hbm.at[pl.ds(rp, QRPT),
                                                    pl.ds(c * F, F)])
            if r < K - 1:
                plsc.subcore_barrier()

    return prop


_prop = _make_prop()


def _head_body(h_ref, w_ref, b_ref, o_ref):
    z = lax.dot_general(h_ref[...], w_ref[...], (((1,), (1,)), ((), ())),
                        preferred_element_type=jnp.float32)
    z = z + b_ref[...]
    m = jnp.max(z, axis=-1, keepdims=True)
    lse = jnp.log(jnp.sum(jnp.exp(z - m), axis=-1, keepdims=True))
    o_ref[...] = z - m - lse


_head = pl.pallas_call(
    _head_body,
    grid=(10,),
    in_specs=[
        pl.BlockSpec((N // 10, D), lambda i: (i, 0)),
        pl.BlockSpec((C, D), lambda i: (0, 0)),
        pl.BlockSpec((1, C), lambda i: (0, 0)),
    ],
    out_specs=pl.BlockSpec((N // 10, C), lambda i: (i, 0)),
    out_shape=jax.ShapeDtypeStruct((N, C), jnp.float32),
)


def kernel(x, edge_index, W, b):
    src = edge_index[0].astype(jnp.int32)
    dst = edge_index[1].astype(jnp.int32)
    pad = EPAD - E
    padv = jnp.full((pad,), PAD_NODE, jnp.int32)
    src = jnp.concatenate([src, padv])
    dst = jnp.concatenate([dst, padv])
    src2 = src.reshape(16, NCHUNK, CH)
    dst3 = dst.reshape(16, NCHUNK, CH)
    xp = jnp.zeros((NPAD, D), jnp.float32).at[:N].set(x)
    (h2,) = _prop(xp, src2, dst3)
    return _head(h2, W, b.reshape(1, C))
